# Initial kernel scaffold; baseline (speedup 1.0000x reference)
import jax
import jax.numpy as jnp
from jax.experimental import pallas as pl


def kernel(x, edge_index, W1, b1, W2, b2):
    raise NotImplementedError("write your pallas kernel here")


# speedup vs baseline: 5.1273x; 5.1273x over previous
import functools

import jax
import jax.numpy as jnp
from jax import lax
from jax.experimental import pallas as pl
from jax.experimental.pallas import tpu as pltpu
from jax.experimental.pallas import tpu_sc as plsc

N = 10000
E = 320000
D = 128

NC = 2
NS = 16
NW = NC * NS
EPW = E // NW
CHUNK = 80
NCHUNK = EPW // CHUNK
NPAD = 10240
ROWS_PER_TILE = NPAD // NS
ZROWS = 128


def _sc_segsum_body(h_hbm, src_hbm, dst_hbm, p0_hbm, p1_hbm,
                    acc, zb, si, di, rb, gsem):
    c = lax.axis_index("c")
    s = lax.axis_index("s")
    w = s * NC + c

    z16 = jnp.zeros((16,), jnp.float32)

    def _zrow(r, carry):
        for q in range(D // 16):
            zb[r, pl.ds(q * 16, 16)] = z16
        return carry

    lax.fori_loop(0, ZROWS, _zrow, 0)
    row0 = s * ROWS_PER_TILE
    for k in range(ROWS_PER_TILE // ZROWS):
        pltpu.sync_copy(zb, acc.at[pl.ds(row0 + k * ZROWS, ZROWS)])
    plsc.subcore_barrier()

    def _chunk(i, carry):
        base = w * EPW + i * CHUNK
        pltpu.sync_copy(src_hbm.at[pl.ds(base, CHUNK)], si)
        pltpu.async_copy(h_hbm.at[si], rb, gsem).wait()
        pltpu.sync_copy(dst_hbm.at[pl.ds(base, CHUNK)], di)
        pltpu.sync_copy(rb, acc.at[di], add=True)
        return carry

    lax.fori_loop(0, NCHUNK, _chunk, 0)
    plsc.subcore_barrier()

    @pl.when(c == 0)
    def _():
        pltpu.sync_copy(acc.at[pl.ds(row0, ROWS_PER_TILE)],
                        p0_hbm.at[pl.ds(row0, ROWS_PER_TILE)])

    @pl.when(c == 1)
    def _():
        pltpu.sync_copy(acc.at[pl.ds(row0, ROWS_PER_TILE)],
                        p1_hbm.at[pl.ds(row0, ROWS_PER_TILE)])


_sc_segsum = functools.partial(
    pl.kernel,
    out_type=(jax.ShapeDtypeStruct((NPAD, D), jnp.float32),
              jax.ShapeDtypeStruct((NPAD, D), jnp.float32)),
    mesh=plsc.VectorSubcoreMesh(core_axis_name="c", subcore_axis_name="s"),
    scratch_types=[
        pltpu.VMEM_SHARED((NPAD, D), jnp.float32),
        pltpu.VMEM((ZROWS, D), jnp.float32),
        pltpu.VMEM((CHUNK,), jnp.int32),
        pltpu.VMEM((CHUNK,), jnp.int32),
        pltpu.VMEM((CHUNK, D), jnp.float32),
        pltpu.SemaphoreType.DMA,
    ],
)(_sc_segsum_body)


BLK = 1000


def _mm_body(x_ref, w_ref, b_ref, o_ref):
    o_ref[...] = jnp.dot(x_ref[...], w_ref[...],
                         preferred_element_type=jnp.float32) + b_ref[...]


def _tc_matmul(x, w, b):
    return pl.pallas_call(
        _mm_body,
        grid=(N // BLK,),
        in_specs=[
            pl.BlockSpec((BLK, D), lambda i: (i, 0)),
            pl.BlockSpec((D, D), lambda i: (0, 0)),
            pl.BlockSpec((1, D), lambda i: (0, 0)),
        ],
        out_specs=pl.BlockSpec((BLK, D), lambda i: (i, 0)),
        out_shape=jax.ShapeDtypeStruct((N, D), jnp.float32),
    )(x, w, b.reshape(1, D))


def _combine_mm_body(p0_ref, p1_ref, skip_ref, w_ref, b_ref, s_ref, h_ref):
    sblk = jnp.maximum(p0_ref[...] + p1_ref[...], 0.0) + skip_ref[...]
    s_ref[...] = sblk
    h_ref[...] = jnp.dot(sblk, w_ref[...],
                         preferred_element_type=jnp.float32) + b_ref[...]


def _tc_combine_matmul(p0, p1, skip, w, b):
    return pl.pallas_call(
        _combine_mm_body,
        grid=(N // BLK,),
        in_specs=[
            pl.BlockSpec((BLK, D), lambda i: (i, 0)),
            pl.BlockSpec((BLK, D), lambda i: (i, 0)),
            pl.BlockSpec((BLK, D), lambda i: (i, 0)),
            pl.BlockSpec((D, D), lambda i: (0, 0)),
            pl.BlockSpec((1, D), lambda i: (0, 0)),
        ],
        out_specs=[
            pl.BlockSpec((BLK, D), lambda i: (i, 0)),
            pl.BlockSpec((BLK, D), lambda i: (i, 0)),
        ],
        out_shape=[
            jax.ShapeDtypeStruct((N, D), jnp.float32),
            jax.ShapeDtypeStruct((N, D), jnp.float32),
        ],
    )(p0, p1, skip, w, b.reshape(1, D))


def _combine_body(p0_ref, p1_ref, skip_ref, o_ref):
    o_ref[...] = jnp.maximum(p0_ref[...] + p1_ref[...], 0.0) + skip_ref[...]


def _tc_combine(p0, p1, skip):
    return pl.pallas_call(
        _combine_body,
        grid=(N // BLK,),
        in_specs=[
            pl.BlockSpec((BLK, D), lambda i: (i, 0)),
            pl.BlockSpec((BLK, D), lambda i: (i, 0)),
            pl.BlockSpec((BLK, D), lambda i: (i, 0)),
        ],
        out_specs=pl.BlockSpec((BLK, D), lambda i: (i, 0)),
        out_shape=jax.ShapeDtypeStruct((N, D), jnp.float32),
    )(p0, p1, skip)


def kernel(x, edge_index, W1, b1, W2, b2):
    src = edge_index[0]
    dst = edge_index[1]
    h1 = _tc_matmul(x, W1, b1)
    p0, p1 = _sc_segsum(h1, src, dst)
    s, h2 = _tc_combine_matmul(p0, p1, x, W2, b2)
    q0, q1 = _sc_segsum(h2, src, dst)
    return _tc_combine(q0, q1, s)
